# baseline (speedup 1.0000x reference)
import jax
import jax.numpy as jnp
from jax import lax
from jax.experimental import pallas as pl
from jax.experimental.pallas import tpu as pltpu
from jax.experimental.pallas import tpu_sc as plsc

_B, _S, _D = 4, 4096, 1024
_NC, _NS = 2, 16
_NW = _NC * _NS
_RPW = _S // _NW
_CHR = 32
_CH = _CHR * _D
_NCHUNK = _RPW // _CHR


def _sc_body(x_hbm, pos_hbm, out_hbm, pos_buf, x_buf):
    wid = lax.axis_index("s") * _NC + lax.axis_index("c")
    base = wid * _RPW * _D
    for ci in range(_NCHUNK):
        poff = base + ci * _CH
        pltpu.sync_copy(pos_hbm.at[pl.ds(poff, _CH)], pos_buf)
        for b in range(_B):
            xoff = b * _S * _D + poff
            pltpu.sync_copy(x_hbm.at[pl.ds(xoff, _CH)], x_buf)

            @plsc.parallel_loop(0, _CH, 16, unroll=8)
            def _add(i):
                x_buf[pl.ds(i, 16)] = x_buf[pl.ds(i, 16)] + pos_buf[pl.ds(i, 16)]

            pltpu.sync_copy(x_buf, out_hbm.at[pl.ds(xoff, _CH)])


def kernel(x, pos_table):
    mesh = plsc.VectorSubcoreMesh(core_axis_name="c", subcore_axis_name="s")
    k = pl.kernel(
        _sc_body,
        out_type=jax.ShapeDtypeStruct((_B * _S * _D,), jnp.float32),
        mesh=mesh,
        scratch_types=[
            pltpu.VMEM((_CH,), jnp.float32),
            pltpu.VMEM((_CH,), jnp.float32),
        ],
    )
    out = k(x.reshape(-1), pos_table.reshape(-1))
    return out.reshape(x.shape)


# speedup vs baseline: 1.2108x; 1.2108x over previous
import jax
import jax.numpy as jnp
from jax import lax
from jax.experimental import pallas as pl
from jax.experimental.pallas import tpu as pltpu
from jax.experimental.pallas import tpu_sc as plsc

_B, _S, _D = 4, 4096, 1024
_NC, _NS = 2, 16
_NW = _NC * _NS
_RPW = _S // _NW
_CHR = 16
_CH = _CHR * _D
_NCHUNK = _RPW // _CHR
_NSTEP = _NCHUNK * _B
_NIN = 3
_NOUT = 2


def _sc_body(x_hbm, pos_hbm, out_hbm, pos0, pos1, in0, in1, in2, outb0, outb1,
             psem0, psem1, xsem0, xsem1, xsem2, osem0, osem1):
    pos_buf = (pos0, pos1)
    in_buf = (in0, in1, in2)
    out_buf = (outb0, outb1)
    psem = (psem0, psem1)
    xsem = (xsem0, xsem1, xsem2)
    osem = (osem0, osem1)
    wid = lax.axis_index("s") * _NC + lax.axis_index("c")
    base = wid * _RPW * _D

    def x_off(step):
        ci, b = step // _B, step % _B
        return b * _S * _D + base + ci * _CH

    pos_dma = [None] * _NCHUNK
    pos_dma[0] = pltpu.async_copy(pos_hbm.at[pl.ds(base, _CH)],
                                  pos_buf[0], psem[0])
    in_dma = [None] * _NSTEP
    out_dma = [None] * _NSTEP
    for s in range(_NIN):
        in_dma[s] = pltpu.async_copy(x_hbm.at[pl.ds(x_off(s), _CH)],
                                     in_buf[s % _NIN], xsem[s % _NIN])

    for s in range(_NSTEP):
        ci, b = s // _B, s % _B
        ri, ro = s % _NIN, s % _NOUT
        if b == 0:
            pos_dma[ci].wait()
            if ci + 1 < _NCHUNK:
                pos_dma[ci + 1] = pltpu.async_copy(
                    pos_hbm.at[pl.ds(base + (ci + 1) * _CH, _CH)],
                    pos_buf[(ci + 1) % 2], psem[(ci + 1) % 2])
        in_dma[s].wait()
        if s >= _NOUT:
            out_dma[s - _NOUT].wait()
        pc = ci % 2

        obuf, ibuf, pbuf = out_buf[ro], in_buf[ri], pos_buf[pc]

        @plsc.parallel_loop(0, _CH, 16, unroll=8)
        def _add(i):
            obuf[pl.ds(i, 16)] = ibuf[pl.ds(i, 16)] + pbuf[pl.ds(i, 16)]

        out_dma[s] = pltpu.async_copy(obuf,
                                      out_hbm.at[pl.ds(x_off(s), _CH)],
                                      osem[ro])
        if s + _NIN < _NSTEP:
            in_dma[s + _NIN] = pltpu.async_copy(
                x_hbm.at[pl.ds(x_off(s + _NIN), _CH)],
                in_buf[(s + _NIN) % _NIN], xsem[(s + _NIN) % _NIN])

    out_dma[_NSTEP - 2].wait()
    out_dma[_NSTEP - 1].wait()


def kernel(x, pos_table):
    mesh = plsc.VectorSubcoreMesh(core_axis_name="c", subcore_axis_name="s")
    k = pl.kernel(
        _sc_body,
        out_type=jax.ShapeDtypeStruct((_B * _S * _D,), jnp.float32),
        mesh=mesh,
        scratch_types=[
            pltpu.VMEM((_CH,), jnp.float32),
            pltpu.VMEM((_CH,), jnp.float32),
            pltpu.VMEM((_CH,), jnp.float32),
            pltpu.VMEM((_CH,), jnp.float32),
            pltpu.VMEM((_CH,), jnp.float32),
            pltpu.VMEM((_CH,), jnp.float32),
            pltpu.VMEM((_CH,), jnp.float32),
            pltpu.SemaphoreType.DMA,
            pltpu.SemaphoreType.DMA,
            pltpu.SemaphoreType.DMA,
            pltpu.SemaphoreType.DMA,
            pltpu.SemaphoreType.DMA,
            pltpu.SemaphoreType.DMA,
            pltpu.SemaphoreType.DMA,
        ],
    )
    out = k(x.reshape(-1), pos_table.reshape(-1))
    return out.reshape(x.shape)
